# Initial kernel scaffold; baseline (speedup 1.0000x reference)
import jax
import jax.numpy as jnp
from jax.experimental import pallas as pl


def kernel(x, edge_index, edge_weight, latent_variables, W0, b0, g0, be0, W1, b1, g1, be1, W2, b2, g2, be2):
    raise NotImplementedError("write your pallas kernel here")


# speedup vs baseline: 9.0695x; 9.0695x over previous
import functools

import jax
import jax.numpy as jnp
from jax import lax
from jax.experimental import pallas as pl
from jax.experimental.pallas import tpu as pltpu
from jax.experimental.pallas import tpu_sc as plsc

N = 10000
E = 320000
D = 128
NC = 2
NS = 16
NW = NC * NS
EPT = E // NW
CH = 80
NCH = EPT // CH
SB = 25
NSB = NCH // SB
NPA = 10240
RPT = NPA // NS
RCP = 128
NP = 16384
DPT = NP // NS

_PERM = [32 * j + (m // 2 if m % 2 == 0 else 16 + m // 2)
         for j in range(4) for m in range(32)]

_mesh = plsc.VectorSubcoreMesh(core_axis_name="c", subcore_axis_name="s",
                               num_cores=NC, num_subcores=NS)


def _deg_body(dst_hbm, ew_hbm, out_hbm, dst_v, ew_v, stage_v, deg_sh, sem):
    c = lax.axis_index("c")
    s = lax.axis_index("s")
    wid = c * NS + s
    for b in range(NSB):
        pltpu.sync_copy(dst_hbm.at[wid, b], dst_v.at[pl.ds(b * SB, SB)])
        pltpu.sync_copy(ew_hbm.at[wid, b], ew_v.at[pl.ds(b * SB, SB)])
    z = jnp.zeros((16,), jnp.float32)
    for j in range(DPT // 16):
        stage_v[pl.ds(j * 16, 16)] = z
    pltpu.sync_copy(stage_v, deg_sh.at[pl.ds(s * DPT, DPT)])
    plsc.subcore_barrier()

    def chunk(i, carry):
        pltpu.async_copy(ew_v.at[i], deg_sh.at[dst_v.at[i]], sem,
                         add=True).wait()
        return carry

    lax.fori_loop(0, NCH, chunk, 0)
    plsc.subcore_barrier()
    pltpu.sync_copy(deg_sh.at[pl.ds(s * DPT, DPT)], stage_v)
    pltpu.sync_copy(stage_v, out_hbm.at[pl.ds(c * NP + s * DPT, DPT)])


_deg_call = functools.partial(
    pl.kernel,
    out_type=jax.ShapeDtypeStruct((NC * NP,), jnp.float32),
    mesh=_mesh,
    scratch_types=[
        pltpu.VMEM((NCH, CH), jnp.int32),
        pltpu.VMEM((NCH, CH), jnp.float32),
        pltpu.VMEM((DPT,), jnp.float32),
        pltpu.VMEM_SHARED((NP,), jnp.float32),
        pltpu.SemaphoreType.DMA,
    ],
    compiler_params=pltpu.CompilerParams(needs_layout_passes=False),
)(_deg_body)


def _agg_body(hws_hbm, src_hbm, dst_hbm, ew_hbm, out_hbm,
              src_v, dst_v, ew_v, rin_v, rout_v, agg_sh, gsems, ssems):
    c = lax.axis_index("c")
    s = lax.axis_index("s")
    wid = c * NS + s

    z = jnp.zeros((16,), jnp.float32)

    def zrow(r, carry):
        for j in range(D // 16):
            rout_v[0, r, pl.ds(j * 16, 16)] = z
        return carry

    lax.fori_loop(0, CH, zrow, 0)
    for k in range(RPT // CH):
        pltpu.sync_copy(rout_v.at[0], agg_sh.at[pl.ds(s * RPT + k * CH, CH)])
    plsc.subcore_barrier()

    msk = jnp.full((16,), -65536, jnp.int32)

    def scale(x, r3, r2):
        bx = lax.broadcast(x, (16,))

        def edge4(ii, carry):
            for u in range(4):
                e = ii * 4 + u
                be = lax.broadcast(e, (16,))
                w = plsc.load_gather(ew_v, [bx, be])
                for j in range(D // 32):
                    v = rin_v[r3, e, pl.ds(16 * j, 16)]
                    ev = plsc.bitcast(lax.shift_left(v, 16), jnp.float32)
                    od = plsc.bitcast(jnp.bitwise_and(v, msk), jnp.float32)
                    rout_v[r2, e, pl.ds(32 * j, 16)] = ev * w
                    rout_v[r2, e, pl.ds(32 * j + 16, 16)] = od * w
            return carry

        lax.fori_loop(0, CH // 4, edge4, 0)

    def wait_g(x, r3):
        pltpu.make_async_copy(hws_hbm.at[src_v.at[x]], rin_v.at[r3],
                              gsems.at[r3]).wait()

    def issue_g(x, r3):
        pltpu.async_copy(hws_hbm.at[src_v.at[x]], rin_v.at[r3], gsems.at[r3])

    def issue_sc(x, r2):
        pltpu.async_copy(rout_v.at[r2], agg_sh.at[dst_v.at[x]], ssems.at[r2],
                         add=True)

    def wait_sc(x, r2):
        pltpu.make_async_copy(rout_v.at[r2], agg_sh.at[dst_v.at[x]],
                              ssems.at[r2]).wait()

    def step(x, wsc, refill):
        r3 = lax.rem(x, 3)
        r2 = lax.rem(x, 2)
        wait_g(x, r3)
        if wsc:
            wait_sc(x - 2, r2)
        scale(x, r3, r2)
        issue_sc(x, r2)
        if refill:
            issue_g(x + 3, r3)

    def generic(x, carry):
        step(x, True, True)
        return carry

    def block(b, carry):
        pltpu.sync_copy(src_hbm.at[wid, b], src_v)
        pltpu.sync_copy(dst_hbm.at[wid, b], dst_v)
        pltpu.sync_copy(ew_hbm.at[wid, b], ew_v)
        for u in range(3):
            issue_g(u, u)
        step(0, False, True)
        step(1, False, True)
        lax.fori_loop(2, SB - 3, generic, 0)
        step(SB - 3, True, False)
        step(SB - 2, True, False)
        step(SB - 1, True, False)
        wait_sc(SB - 2, lax.rem(SB - 2, 2))
        wait_sc(SB - 1, lax.rem(SB - 1, 2))
        return carry

    lax.fori_loop(0, NSB, block, 0)
    plsc.subcore_barrier()
    for k in range(RPT // CH):
        base = s * RPT + k * CH
        pltpu.sync_copy(agg_sh.at[pl.ds(base, CH)], rout_v.at[0])
        pltpu.sync_copy(rout_v.at[0], out_hbm.at[pl.ds(c * NPA + base, CH)])


_agg_call = functools.partial(
    pl.kernel,
    out_type=jax.ShapeDtypeStruct((NC * NPA, D), jnp.float32),
    mesh=_mesh,
    scratch_types=[
        pltpu.VMEM((SB, CH), jnp.int32),
        pltpu.VMEM((SB, CH), jnp.int32),
        pltpu.VMEM((SB, CH), jnp.float32),
        pltpu.VMEM((3, CH, D // 2), jnp.int32),
        pltpu.VMEM((2, CH, D), jnp.float32),
        pltpu.VMEM_SHARED((NPA, D), jnp.float32),
        pltpu.SemaphoreType.DMA((3,)),
        pltpu.SemaphoreType.DMA((2,)),
    ],
    compiler_params=pltpu.CompilerParams(needs_layout_passes=False,
                                         use_tc_tiling_on_sc=False),
)(_agg_body)


def _pre_body(deg_ref, lat_ref, w_ref, dis_ref, hws_ref):
    deg = deg_ref[0:N] + deg_ref[NP:NP + N]
    dis = jnp.where(deg > 0, lax.rsqrt(jnp.maximum(deg, 1e-12)), 0.0)
    dis2 = dis[:, None]
    dis_ref[...] = dis2
    hw = jnp.dot(lat_ref[...], w_ref[...], preferred_element_type=jnp.float32)
    hws_ref[...] = (hw * dis2).astype(jnp.bfloat16)


def _pre_call(deg, lat, w):
    return pl.pallas_call(
        _pre_body,
        out_shape=(
            jax.ShapeDtypeStruct((N, 1), jnp.float32),
            jax.ShapeDtypeStruct((N, D), jnp.bfloat16),
        ),
    )(deg, lat, w)


def _make_epi_body(relu, matmul):
    def body(a_ref, dis_ref, b_ref, g_ref, be_ref, lat_ref, *rest):
        if matmul:
            w_ref, out_ref = rest
        else:
            (out_ref,) = rest
        dis2 = dis_ref[...]
        sacc = (a_ref[0:N] + a_ref[NPA:NPA + N]) * dis2 + b_ref[...]
        if relu:
            sacc = jnp.maximum(sacc, 0.0)
        h = sacc + lat_ref[...]
        m = jnp.mean(h, axis=0, keepdims=True)
        hc = h - m
        v = jnp.mean(hc * hc, axis=0, keepdims=True)
        hn = g_ref[...] * hc / jnp.sqrt(v + 1e-5) + be_ref[...]
        if matmul:
            hw = jnp.dot(hn, w_ref[...], preferred_element_type=jnp.float32)
            out_ref[...] = (hw * dis2).astype(jnp.bfloat16)
        else:
            out_ref[...] = hn

    return body


def _epi_mm(relu, a, dis, b, g, be, lat, w):
    return pl.pallas_call(
        _make_epi_body(relu, True),
        out_shape=jax.ShapeDtypeStruct((N, D), jnp.bfloat16),
    )(a, dis, b, g, be, lat, w)


def _epi_last(relu, a, dis, b, g, be, lat):
    return pl.pallas_call(
        _make_epi_body(relu, False),
        out_shape=jax.ShapeDtypeStruct((N, D), jnp.float32),
    )(a, dis, b, g, be, lat)


def kernel(x, edge_index, edge_weight, latent_variables,
           W0, b0, g0, be0, W1, b1, g1, be1, W2, b2, g2, be2):
    src = edge_index[0].reshape(NW, NSB, SB, CH)
    dst = edge_index[1].reshape(NW, NSB, SB, CH)
    ew = edge_weight.reshape(NW, NSB, SB, CH)
    lat = latent_variables
    perm = jnp.asarray(_PERM, dtype=jnp.int32)
    W0p = W0[:, perm]
    W1p = W1[:, perm]
    W2p = W2[:, perm]

    deg = _deg_call(dst, ew)
    dis, hws = _pre_call(deg, lat, W0p)

    b0r, g0r, be0r = b0.reshape(1, D), g0.reshape(1, D), be0.reshape(1, D)
    b1r, g1r, be1r = b1.reshape(1, D), g1.reshape(1, D), be1.reshape(1, D)
    b2r, g2r, be2r = b2.reshape(1, D), g2.reshape(1, D), be2.reshape(1, D)

    def pack32(hb):
        return lax.bitcast_convert_type(hb.reshape(N, D // 2, 2), jnp.int32)

    a = _agg_call(pack32(hws), src, dst, ew)
    hws = _epi_mm(True, a, dis, b0r, g0r, be0r, lat, W1p)
    a = _agg_call(pack32(hws), src, dst, ew)
    hws = _epi_mm(False, a, dis, b1r, g1r, be1r, lat, W2p)
    a = _agg_call(pack32(hws), src, dst, ew)
    return _epi_last(True, a, dis, b2r, g2r, be2r, lat)


# speedup vs baseline: 16.4966x; 1.8189x over previous
import functools

import jax
import jax.numpy as jnp
from jax import lax
from jax.experimental import pallas as pl
from jax.experimental.pallas import tpu as pltpu
from jax.experimental.pallas import tpu_sc as plsc

N = 10000
E = 320000
D = 128
NC = 2
NS = 16
NW = NC * NS
EPT = E // NW
CH = 80
NCH = EPT // CH
SB = 25
NSB = NCH // SB
NPA = 10240
RPT = NPA // NS
RCP = 128
NP = 16384
DPT = NP // NS

_mesh = plsc.VectorSubcoreMesh(core_axis_name="c", subcore_axis_name="s",
                               num_cores=NC, num_subcores=NS)


def _deg_body(dst_hbm, ew_hbm, out_hbm, dst_v, ew_v, stage_v, deg_sh, sem):
    c = lax.axis_index("c")
    s = lax.axis_index("s")
    wid = c * NS + s
    for b in range(NSB):
        pltpu.sync_copy(dst_hbm.at[wid, b], dst_v.at[pl.ds(b * SB, SB)])
        pltpu.sync_copy(ew_hbm.at[wid, b], ew_v.at[pl.ds(b * SB, SB)])
    z = jnp.zeros((16,), jnp.float32)
    for j in range(DPT // 16):
        stage_v[pl.ds(j * 16, 16)] = z
    pltpu.sync_copy(stage_v, deg_sh.at[pl.ds(s * DPT, DPT)])
    plsc.subcore_barrier()

    def chunk(i, carry):
        pltpu.async_copy(ew_v.at[i], deg_sh.at[dst_v.at[i]], sem,
                         add=True).wait()
        return carry

    lax.fori_loop(0, NCH, chunk, 0)
    plsc.subcore_barrier()
    pltpu.sync_copy(deg_sh.at[pl.ds(s * DPT, DPT)], stage_v)
    pltpu.sync_copy(stage_v, out_hbm.at[pl.ds(c * NP + s * DPT, DPT)])


_deg_call = functools.partial(
    pl.kernel,
    out_type=jax.ShapeDtypeStruct((NC * NP,), jnp.float32),
    mesh=_mesh,
    scratch_types=[
        pltpu.VMEM((NCH, CH), jnp.int32),
        pltpu.VMEM((NCH, CH), jnp.float32),
        pltpu.VMEM((DPT,), jnp.float32),
        pltpu.VMEM_SHARED((NP,), jnp.float32),
        pltpu.SemaphoreType.DMA,
    ],
    compiler_params=pltpu.CompilerParams(needs_layout_passes=False),
)(_deg_body)


def _agg_body(hws_hbm, src_hbm, dst_hbm, ew_hbm, out_hbm,
              src_v, dst_v, ew_v, rows_v, agg_sh, gsems, ssems):
    c = lax.axis_index("c")
    s = lax.axis_index("s")
    wid = c * NS + s

    z = jnp.zeros((16,), jnp.float32)

    def zrow(r, carry):
        for j in range(D // 16):
            rows_v[0, r, pl.ds(j * 16, 16)] = z
        return carry

    lax.fori_loop(0, CH, zrow, 0)
    for k in range(RPT // CH):
        pltpu.sync_copy(rows_v.at[0], agg_sh.at[pl.ds(s * RPT + k * CH, CH)])
    plsc.subcore_barrier()

    def scale_scatter(i, rv):
        bi = lax.broadcast(i, (16,))

        def edge4(ii, carry):
            for u in range(4):
                e = ii * 4 + u
                be = lax.broadcast(e, (16,))
                w = plsc.load_gather(ew_v, [bi, be])
                for j in range(D // 16):
                    rv[e, pl.ds(j * 16, 16)] = rv[e, pl.ds(j * 16, 16)] * w
            return carry

        lax.fori_loop(0, CH // 4, edge4, 0)
        pltpu.sync_copy(rv, agg_sh.at[dst_v.at[i]], add=True)

    def scale(i, rv):
        bi = lax.broadcast(i, (16,))

        def edge4(ii, carry):
            for u in range(4):
                e = ii * 4 + u
                be = lax.broadcast(e, (16,))
                w = plsc.load_gather(ew_v, [bi, be])
                for j in range(D // 16):
                    rv[e, pl.ds(j * 16, 16)] = rv[e, pl.ds(j * 16, 16)] * w
            return carry

        lax.fori_loop(0, CH // 4, edge4, 0)

    def wait_g(x, r):
        pltpu.make_async_copy(hws_hbm.at[src_v.at[x]], rows_v.at[r],
                              gsems.at[r]).wait()

    def issue_g(x, r):
        pltpu.async_copy(hws_hbm.at[src_v.at[x]], rows_v.at[r], gsems.at[r])

    def issue_sc(x, r):
        pltpu.async_copy(rows_v.at[r], agg_sh.at[dst_v.at[x]], ssems.at[r],
                         add=True)

    def wait_sc(x, r):
        pltpu.make_async_copy(rows_v.at[r], agg_sh.at[dst_v.at[x]],
                              ssems.at[r]).wait()

    def step(x, r, refill):
        q = (r + 2) % 3
        wait_g(x, r)
        scale(x, rows_v.at[r])
        issue_sc(x, r)
        wait_sc(x, q)
        if refill:
            issue_g(x + 2, q)

    def triple(i, carry):
        x = 1 + 3 * i
        for u in range(3):
            step(x + u, (1 + u) % 3, True)
        return carry

    def block(b, carry):
        pltpu.sync_copy(src_hbm.at[wid, b], src_v)
        pltpu.sync_copy(dst_hbm.at[wid, b], dst_v)
        pltpu.sync_copy(ew_hbm.at[wid, b], ew_v)
        for u in range(3):
            issue_g(u, u)
        wait_g(0, 0)
        scale(0, rows_v.at[0])
        issue_sc(0, 0)
        lax.fori_loop(0, 7, triple, 0)
        step(22, 1, True)
        step(23, 2, False)
        step(24, 0, False)
        wait_sc(24, 0)
        return carry

    lax.fori_loop(0, NSB, block, 0)
    plsc.subcore_barrier()
    for k in range(RPT // CH):
        base = s * RPT + k * CH
        pltpu.sync_copy(agg_sh.at[pl.ds(base, CH)], rows_v.at[0])
        pltpu.sync_copy(rows_v.at[0], out_hbm.at[pl.ds(c * NPA + base, CH)])


_agg_call = functools.partial(
    pl.kernel,
    out_type=jax.ShapeDtypeStruct((NC * NPA, D), jnp.float32),
    mesh=_mesh,
    scratch_types=[
        pltpu.VMEM((SB, CH), jnp.int32),
        pltpu.VMEM((SB, CH), jnp.int32),
        pltpu.VMEM((SB, CH), jnp.float32),
        pltpu.VMEM((3, CH, D), jnp.float32),
        pltpu.VMEM_SHARED((NPA, D), jnp.float32),
        pltpu.SemaphoreType.DMA((3,)),
        pltpu.SemaphoreType.DMA((3,)),
    ],
    compiler_params=pltpu.CompilerParams(needs_layout_passes=False),
)(_agg_body)


def _pre_body(deg_ref, lat_ref, w_ref, dis_ref, hws_ref):
    deg = deg_ref[0:N] + deg_ref[NP:NP + N]
    dis = jnp.where(deg > 0, lax.rsqrt(jnp.maximum(deg, 1e-12)), 0.0)
    dis2 = dis[:, None]
    dis_ref[...] = dis2
    hw = jnp.dot(lat_ref[...], w_ref[...], preferred_element_type=jnp.float32)
    hws_ref[...] = hw * dis2


def _pre_call(deg, lat, w):
    return pl.pallas_call(
        _pre_body,
        out_shape=(
            jax.ShapeDtypeStruct((N, 1), jnp.float32),
            jax.ShapeDtypeStruct((N, D), jnp.float32),
        ),
    )(deg, lat, w)


def _make_epi_body(relu, matmul):
    def body(a_ref, dis_ref, b_ref, g_ref, be_ref, lat_ref, *rest):
        if matmul:
            w_ref, out_ref = rest
        else:
            (out_ref,) = rest
        dis2 = dis_ref[...]
        sacc = (a_ref[0:N] + a_ref[NPA:NPA + N]) * dis2 + b_ref[...]
        if relu:
            sacc = jnp.maximum(sacc, 0.0)
        h = sacc + lat_ref[...]
        m = jnp.mean(h, axis=0, keepdims=True)
        hc = h - m
        v = jnp.mean(hc * hc, axis=0, keepdims=True)
        hn = g_ref[...] * hc / jnp.sqrt(v + 1e-5) + be_ref[...]
        if matmul:
            hw = jnp.dot(hn, w_ref[...], preferred_element_type=jnp.float32)
            out_ref[...] = hw * dis2
        else:
            out_ref[...] = hn

    return body


def _epi_mm(relu, a, dis, b, g, be, lat, w):
    return pl.pallas_call(
        _make_epi_body(relu, True),
        out_shape=jax.ShapeDtypeStruct((N, D), jnp.float32),
    )(a, dis, b, g, be, lat, w)


def _epi_last(relu, a, dis, b, g, be, lat):
    return pl.pallas_call(
        _make_epi_body(relu, False),
        out_shape=jax.ShapeDtypeStruct((N, D), jnp.float32),
    )(a, dis, b, g, be, lat)


def kernel(x, edge_index, edge_weight, latent_variables,
           W0, b0, g0, be0, W1, b1, g1, be1, W2, b2, g2, be2):
    src = edge_index[0].reshape(NW, NSB, SB, CH)
    dst = edge_index[1].reshape(NW, NSB, SB, CH)
    ew = edge_weight.reshape(NW, NSB, SB, CH)
    lat = latent_variables

    deg = _deg_call(dst, ew)
    dis, hws = _pre_call(deg, lat, W0)

    b0r, g0r, be0r = b0.reshape(1, D), g0.reshape(1, D), be0.reshape(1, D)
    b1r, g1r, be1r = b1.reshape(1, D), g1.reshape(1, D), be1.reshape(1, D)
    b2r, g2r, be2r = b2.reshape(1, D), g2.reshape(1, D), be2.reshape(1, D)

    a = _agg_call(hws, src, dst, ew)
    hws = _epi_mm(True, a, dis, b0r, g0r, be0r, lat, W1)
    a = _agg_call(hws, src, dst, ew)
    hws = _epi_mm(False, a, dis, b1r, g1r, be1r, lat, W2)
    a = _agg_call(hws, src, dst, ew)
    return _epi_last(True, a, dis, b2r, g2r, be2r, lat)
